# Initial kernel scaffold; baseline (speedup 1.0000x reference)
import jax
import jax.numpy as jnp
from jax.experimental import pallas as pl


def kernel(x, shared_Wg, shared_Wu, shared_Wd, Wg, Wu, Wd, Wr, br):
    raise NotImplementedError("write your pallas kernel here")


# speedup vs baseline: 2.0209x; 2.0209x over previous
import functools

import jax
import jax.numpy as jnp
from jax import lax
from jax.experimental import pallas as pl
from jax.experimental.pallas import tpu as pltpu

H = 1024
FF = 512
NE = 64
NS = 1
NR = NE - NS
TOPK = 2
T = 2048

BM = 64
MAX_RBLK = 128
SH_BLK = T // BM
NBLK = MAX_RBLK + SH_BLK
RPAD = MAX_RBLK * BM
NROWS = RPAD + T


def _router_body(x_ref, wrt_ref, br_ref, wr_ref, s_out, i_out, z_out):
    i = pl.program_id(0)
    xb = x_ref[...]
    logits = jnp.dot(xb, wrt_ref[...],
                     preferred_element_type=jnp.float32)
    logits = logits + br_ref[0:1, :NE]
    col = lax.broadcasted_iota(jnp.int32, logits.shape, 1)
    valid = col < NR
    probs = jnp.where(valid, jax.nn.sigmoid(logits), -1.0)
    m1 = jnp.max(probs, axis=1, keepdims=True)
    i1 = jnp.min(jnp.where(probs == m1, col, NE), axis=1, keepdims=True)
    probs2 = jnp.where(col == i1, -1.0, probs)
    m2 = jnp.max(probs2, axis=1, keepdims=True)
    i2 = jnp.min(jnp.where(probs2 == m2, col, NE), axis=1, keepdims=True)
    den = m1 + m2
    s1 = m1 / den
    s2 = m2 / den
    c128 = lax.broadcasted_iota(jnp.int32, (xb.shape[0], 128), 1)
    zf = jnp.zeros_like(c128, dtype=jnp.float32)
    zi = jnp.zeros_like(c128)
    s_out[...] = jnp.where(c128 == 0, s1, jnp.where(c128 == 1, s2, zf))
    i_out[...] = jnp.where(c128 == 0, i1, jnp.where(c128 == 1, i2, zi))

    @pl.when(i == 0)
    def _():
        w = wr_ref[...]
        lg = jnp.log(jnp.sum(jnp.exp(w), axis=1, keepdims=True))
        rio = lax.broadcasted_iota(jnp.int32, lg.shape, 0)
        tot = jnp.sum(jnp.where(rio < NR, lg, 0.0))
        z_out[...] = jnp.full((8, 128), 0.001 * tot / NR)


def _router(x2, Wr, br):
    TB = 256
    wr_pad = jnp.zeros((NE, H), jnp.float32).at[:NR].set(Wr)
    wrt = wr_pad.T
    br_pad = jnp.zeros((8, 128), jnp.float32).at[0, :NR].set(br)
    s_out, i_out, z_out = pl.pallas_call(
        _router_body,
        grid=(T // TB,),
        in_specs=[
            pl.BlockSpec((TB, H), lambda i: (i, 0)),
            pl.BlockSpec((H, NE), lambda i: (0, 0)),
            pl.BlockSpec((8, 128), lambda i: (0, 0)),
            pl.BlockSpec((NE, H), lambda i: (0, 0)),
        ],
        out_specs=[
            pl.BlockSpec((TB, 128), lambda i: (i, 0)),
            pl.BlockSpec((TB, 128), lambda i: (i, 0)),
            pl.BlockSpec((8, 128), lambda i: (0, 0)),
        ],
        out_shape=[
            jax.ShapeDtypeStruct((T, 128), jnp.float32),
            jax.ShapeDtypeStruct((T, 128), jnp.int32),
            jax.ShapeDtypeStruct((8, 128), jnp.float32),
        ],
    )(x2, wrt, br_pad, wr_pad)
    return s_out[:, :TOPK], i_out[:, :TOPK], z_out[0, 0]


def _gemm_body(s_ref, xg_ref, wg_ref, wu_ref, wd_ref, sc_ref, out_ref):
    xb = xg_ref[...]
    dn = (((1,), (1,)), ((), ()))
    g = lax.dot_general(xb, wg_ref[0], dn,
                        preferred_element_type=jnp.float32)
    u = lax.dot_general(xb, wu_ref[0], dn,
                        preferred_element_type=jnp.float32)
    h = (g * jax.nn.sigmoid(g)) * u
    y = lax.dot_general(h, wd_ref[0], dn,
                        preferred_element_type=jnp.float32)
    s = sc_ref[0, 0, :BM]
    out_ref[...] = y * s[:, None]


def _grouped_gemm(xg, Wg_all, Wu_all, Wd_all, scores_blk, blk_expert):
    grid_spec = pltpu.PrefetchScalarGridSpec(
        num_scalar_prefetch=1,
        grid=(NBLK,),
        in_specs=[
            pl.BlockSpec((BM, H), lambda i, s: (i, 0)),
            pl.BlockSpec((1, FF, H), lambda i, s: (s[i], 0, 0)),
            pl.BlockSpec((1, FF, H), lambda i, s: (s[i], 0, 0)),
            pl.BlockSpec((1, H, FF), lambda i, s: (s[i], 0, 0)),
            pl.BlockSpec((1, 1, 128), lambda i, s: (i, 0, 0)),
        ],
        out_specs=pl.BlockSpec((BM, H), lambda i, s: (i, 0)),
    )
    return pl.pallas_call(
        _gemm_body,
        grid_spec=grid_spec,
        out_shape=jax.ShapeDtypeStruct((NROWS, H), jnp.float32),
    )(blk_expert, xg, Wg_all, Wu_all, Wd_all, scores_blk)


def kernel(x, shared_Wg, shared_Wu, shared_Wd, Wg, Wu, Wd, Wr, br):
    Bx, Tx, C = x.shape
    x2 = x.reshape(Tx, C)

    scores2, idx2, zloss = _router(x2, Wr, br)

    e_flat = idx2.reshape(-1)
    s_flat = scores2.reshape(-1)
    order = jnp.argsort(e_flat)
    e_sorted = e_flat[order]
    tok_sorted = (order // TOPK).astype(jnp.int32)
    s_sorted = s_flat[order]
    gsz = jnp.bincount(e_sorted, length=NR)
    blocks = (gsz + BM - 1) // BM
    cum_blocks = jnp.cumsum(blocks)
    off = (cum_blocks - blocks) * BM
    gstart = jnp.cumsum(gsz) - gsz
    rank = jnp.arange(T * TOPK, dtype=jnp.int32) - gstart[e_sorted]
    pos = (off[e_sorted] + rank).astype(jnp.int32)
    idx_all = jnp.zeros((NROWS,), jnp.int32).at[pos].set(tok_sorted)
    idx_all = idx_all.at[RPAD:].set(jnp.arange(T, dtype=jnp.int32))
    scores_all = jnp.zeros((NROWS,), jnp.float32).at[pos].set(s_sorted)
    scores_all = scores_all.at[RPAD:].set(1.0)
    inv = jnp.zeros((T * TOPK,), jnp.int32).at[order].set(pos)
    p0 = inv[0::2]
    p1 = inv[1::2]
    blk_expert = jnp.searchsorted(
        cum_blocks, jnp.arange(MAX_RBLK, dtype=jnp.int32), side='right'
    ).astype(jnp.int32)
    blk_expert = jnp.concatenate(
        [blk_expert, jnp.full((SH_BLK,), NR, jnp.int32)])
    scores_blk = jnp.zeros((NBLK, 1, 128), jnp.float32).at[:, 0, :BM].set(
        scores_all.reshape(NBLK, BM))

    Wg_all = jnp.concatenate([Wg, shared_Wg], axis=0)
    Wu_all = jnp.concatenate([Wu, shared_Wu], axis=0)
    Wd_all = jnp.concatenate([Wd, shared_Wd], axis=0)

    xg = x2[idx_all]

    out_all = _grouped_gemm(xg, Wg_all, Wu_all, Wd_all, scores_blk, blk_expert)

    final2 = out_all[RPAD:] + out_all[p0] + out_all[p1]
    return final2.reshape(Bx, Tx, C), zloss
